# baseline (speedup 1.0000x reference)
import functools

import jax
import jax.numpy as jnp
from jax import lax
from jax.experimental import pallas as pl
from jax.experimental.pallas import tpu as pltpu
from jax.experimental.pallas import tpu_sc as plsc

N = 10000
N2 = 10240
D = 128
E = 320000
R = 2
NB = 30

NC = 2
NS = 16
NW = NC * NS
CHUNK = 120
NCHUNK = 84
E2 = NW * NCHUNK * CHUNK
ROWS_PT = N2 // NS
STAGES = (120, 120, 120, 120, 120, 40)

BN = 1024
NBLK = N2 // BN


def _prep_body(xin_ref, att_ref, basis_ref, gwt_ref, root_ref, bias_ref,
               y_ref, init_ref, *, relu_sum):
    if relu_sum:
        x = jax.nn.relu(xin_ref[0] + xin_ref[1])
    else:
        x = xin_ref[0]
    dot = functools.partial(
        jax.lax.dot_general, dimension_numbers=(((1,), (0,)), ((), ())),
        preferred_element_type=jnp.float32, precision=lax.Precision.HIGHEST)
    w = dot(att_ref[...], basis_ref[...]).reshape(R, D, D)
    s = jax.nn.sigmoid(dot(x, gwt_ref[...]))
    for r in range(R):
        y_ref[r] = s[:, r:r + 1] * dot(x, w[r])
    init_ref[0] = dot(x, root_ref[...]) + bias_ref[...]
    init_ref[1] = jnp.zeros((BN, D), jnp.float32)


def _make_prep(relu_sum, nx):
    return pl.pallas_call(
        functools.partial(_prep_body, relu_sum=relu_sum),
        grid=(NBLK,),
        in_specs=[
            pl.BlockSpec((nx, BN, D), lambda n: (0, n, 0)),
            pl.BlockSpec((R, NB), lambda n: (0, 0)),
            pl.BlockSpec((NB, D * D), lambda n: (0, 0)),
            pl.BlockSpec((D, R), lambda n: (0, 0)),
            pl.BlockSpec((D, D), lambda n: (0, 0)),
            pl.BlockSpec((1, D), lambda n: (0, 0)),
        ],
        out_specs=[
            pl.BlockSpec((R, BN, D), lambda n: (0, n, 0)),
            pl.BlockSpec((NC, BN, D), lambda n: (0, n, 0)),
        ],
        out_shape=[
            jax.ShapeDtypeStruct((R, N2, D), jnp.float32),
            jax.ShapeDtypeStruct((NC, N2, D), jnp.float32),
        ],
    )


_prep1 = _make_prep(relu_sum=False, nx=1)
_prep2 = _make_prep(relu_sum=True, nx=NC)


def _gidx_body(j_ref, et_ref, g_ref):
    g_ref[...] = et_ref[...] * N2 + j_ref[...]


_gidx_call = pl.pallas_call(
    _gidx_body,
    out_shape=jax.ShapeDtypeStruct((E // 128, 128), jnp.int32),
)


def _fin_body(p_ref, out_ref):
    out_ref[...] = p_ref[0] + p_ref[1]


_fin_call = pl.pallas_call(
    _fin_body,
    grid=(NBLK,),
    in_specs=[pl.BlockSpec((NC, BN, D), lambda n: (0, n, 0))],
    out_specs=pl.BlockSpec((BN, D), lambda n: (n, 0)),
    out_shape=jax.ShapeDtypeStruct((N2, D), jnp.float32),
)


_sc_mesh = plsc.VectorSubcoreMesh(core_axis_name="c", subcore_axis_name="s")


@functools.partial(
    pl.kernel,
    out_type=jax.ShapeDtypeStruct((NC, N2, D), jnp.float32),
    mesh=_sc_mesh,
    scratch_types=[
        pltpu.VMEM_SHARED((N2, D), jnp.float32),
        pltpu.VMEM((3, CHUNK, D), jnp.float32),
        pltpu.VMEM((3, 1, CHUNK), jnp.int32),
        pltpu.VMEM((6, 1, CHUNK), jnp.int32),
        [pltpu.SemaphoreType.DMA] * 3,
        [pltpu.SemaphoreType.DMA] * 3,
        [pltpu.SemaphoreType.DMA] * 3,
        [pltpu.SemaphoreType.DMA] * 6,
    ],
)
def _sc_edges(y_hbm, gidx_hbm, didx_hbm, init_hbm, out_hbm,
              acc, rows, gbuf, dbuf, semG, semS, semIg, semId):
    cid = lax.axis_index("c")
    sid = lax.axis_index("s")
    base = (cid * NS + sid) * NCHUNK

    def g_at(j):
        return gbuf.at[j, 0]

    def d_at(j):
        return dbuf.at[j, 0]

    row0 = sid * ROWS_PT
    for sz in STAGES:
        pltpu.sync_copy(init_hbm.at[cid, pl.ds(row0, sz)],
                        rows.at[0, pl.ds(0, sz)])
        pltpu.sync_copy(rows.at[0, pl.ds(0, sz)], acc.at[pl.ds(row0, sz)])
        row0 += sz
    plsc.subcore_barrier()

    pltpu.sync_copy(gidx_hbm.at[base + 0], gbuf.at[0])
    pltpu.sync_copy(didx_hbm.at[base + 0], dbuf.at[0])
    pltpu.sync_copy(gidx_hbm.at[base + 1], gbuf.at[1])
    pltpu.sync_copy(didx_hbm.at[base + 1], dbuf.at[1])
    pltpu.async_copy(gidx_hbm.at[base + 2], gbuf.at[2], semIg[2])
    pltpu.async_copy(didx_hbm.at[base + 2], dbuf.at[2], semId[2])
    pltpu.async_copy(y_hbm.at[g_at(0)], rows.at[0], semG[0])
    pltpu.async_copy(y_hbm.at[g_at(1)], rows.at[1], semG[1])

    def do_slot(r6, i):
        c = 6 * r6 + i
        i3, i6 = i % 3, i % 6
        j3, j6 = (i + 2) % 3, (i + 3) % 6
        pltpu.make_async_copy(y_hbm.at[g_at(i3)], rows.at[i3],
                              semG[i3]).wait()
        pltpu.async_copy(rows.at[i3], acc.at[d_at(i6)], semS[i3], add=True)

        @pl.when(jnp.logical_and(c >= 1, c + 2 <= NCHUNK - 1))
        def _():
            pltpu.make_async_copy(rows.at[j3], acc.at[d_at((i + 5) % 6)],
                                  semS[j3]).wait()

        @pl.when(c + 2 <= NCHUNK - 1)
        def _():
            pltpu.make_async_copy(gidx_hbm.at[base], gbuf.at[j3],
                                  semIg[j3]).wait()
            pltpu.make_async_copy(didx_hbm.at[base], dbuf.at[(i + 2) % 6],
                                  semId[(i + 2) % 6]).wait()
            pltpu.async_copy(y_hbm.at[g_at(j3)], rows.at[j3], semG[j3])

        @pl.when(c + 3 <= NCHUNK - 1)
        def _():
            pltpu.async_copy(gidx_hbm.at[base + c + 3], gbuf.at[i3],
                             semIg[i3])
            pltpu.async_copy(didx_hbm.at[base + c + 3], dbuf.at[j6],
                             semId[j6])

    def round6(r6, carry):
        for i in range(6):
            do_slot(r6, i)
        return carry

    lax.fori_loop(0, NCHUNK // 6, round6, 0, unroll=False)
    for i in range(3):
        pltpu.make_async_copy(rows.at[i], acc.at[d_at(i)], semS[i]).wait()
    plsc.subcore_barrier()

    row0 = sid * ROWS_PT
    for sz in STAGES:
        pltpu.sync_copy(acc.at[pl.ds(row0, sz)], rows.at[0, pl.ds(0, sz)])
        pltpu.sync_copy(rows.at[0, pl.ds(0, sz)],
                        out_hbm.at[cid, pl.ds(row0, sz)])
        row0 += sz


def _layer(xin, gidx3, didx3, att, basis, gwt, root, bias, relu_sum):
    prep = _prep2 if relu_sum else _prep1
    y, init = prep(xin, att, basis, gwt, root, bias)
    return _sc_edges(y.reshape(R * N2, D), gidx3, didx3, init)


def kernel(meeting_utterance_enc_hidden_states, adj_coos, edge_types,
           basis1, att1, gate1, root1, bias1,
           basis2, att2, gate2, root2, bias2):
    x = jnp.pad(meeting_utterance_enc_hidden_states,
                ((0, 0), (0, N2 - N), (0, 0)))
    i_idx = adj_coos[0, 0]
    j_idx = adj_coos[0, 1]
    et = edge_types[0]

    gidx = _gidx_call(j_idx.reshape(E // 128, 128), et.reshape(E // 128, 128))
    pad_dst = N + jnp.arange(E2 - E, dtype=jnp.int32) % (N2 - N)
    pad_src = jnp.arange(E2 - E, dtype=jnp.int32) % N2
    gidx3 = jnp.concatenate([gidx.reshape(E),
                             pad_src]).reshape(NW * NCHUNK, 1, CHUNK)
    didx3 = jnp.concatenate([i_idx, pad_dst]).reshape(NW * NCHUNK, 1, CHUNK)

    gwt1 = gate1[:, :, 0].T
    gwt2 = gate2[:, :, 0].T
    bias1_2d = bias1.reshape(1, D)
    bias2_2d = bias2.reshape(1, D)

    p1 = _layer(x, gidx3, didx3, att1, basis1.reshape(NB, D * D),
                gwt1, root1, bias1_2d, relu_sum=False)
    p2 = _layer(p1, gidx3, didx3, att2, basis2.reshape(NB, D * D),
                gwt2, root2, bias2_2d, relu_sum=True)
    return _fin_call(p2)[:N]


# speedup vs baseline: 1.0994x; 1.0994x over previous
import functools

import jax
import jax.numpy as jnp
from jax import lax
from jax.experimental import pallas as pl
from jax.experimental.pallas import tpu as pltpu
from jax.experimental.pallas import tpu_sc as plsc

N = 10000
N2 = 10240
D = 128
E = 320000
R = 2
NB = 30

NC = 2
NS = 16
NW = NC * NS
CHUNK = 120
NCHUNK = 84
E2 = NW * NCHUNK * CHUNK
ROWS_PT = N2 // NS
STAGES = (120, 120, 120, 120, 120, 40)

BN = 1024
NBLK = N2 // BN


def _w_body(att_ref, basis_ref, w_ref):
    w_ref[...] = jax.lax.dot_general(
        att_ref[...], basis_ref[...], (((1,), (0,)), ((), ())),
        preferred_element_type=jnp.float32, precision=lax.Precision.HIGHEST)


_w_call = pl.pallas_call(
    _w_body,
    out_shape=jax.ShapeDtypeStruct((R, D * D), jnp.float32),
)


def _prep_body(xin_ref, w_ref, gwt_ref, root_ref, bias_ref, y_ref, init_ref,
               *, relu_sum):
    if relu_sum:
        x = jax.nn.relu(xin_ref[0] + xin_ref[1])
    else:
        x = xin_ref[0]
    dot = functools.partial(
        jax.lax.dot_general, dimension_numbers=(((1,), (0,)), ((), ())),
        preferred_element_type=jnp.float32, precision=lax.Precision.HIGHEST)
    s = jax.nn.sigmoid(dot(x, gwt_ref[...]))
    for r in range(R):
        y_ref[r] = s[:, r:r + 1] * dot(x, w_ref[r])
    init_ref[0] = dot(x, root_ref[...]) + bias_ref[...]
    init_ref[1] = jnp.zeros((BN, D), jnp.float32)


def _make_prep(relu_sum, nx):
    return pl.pallas_call(
        functools.partial(_prep_body, relu_sum=relu_sum),
        grid=(NBLK,),
        in_specs=[
            pl.BlockSpec((nx, BN, D), lambda n: (0, n, 0)),
            pl.BlockSpec((R, D, D), lambda n: (0, 0, 0)),
            pl.BlockSpec((D, R), lambda n: (0, 0)),
            pl.BlockSpec((D, D), lambda n: (0, 0)),
            pl.BlockSpec((1, D), lambda n: (0, 0)),
        ],
        out_specs=[
            pl.BlockSpec((R, BN, D), lambda n: (0, n, 0)),
            pl.BlockSpec((NC, BN, D), lambda n: (0, n, 0)),
        ],
        out_shape=[
            jax.ShapeDtypeStruct((R, N2, D), jnp.float32),
            jax.ShapeDtypeStruct((NC, N2, D), jnp.float32),
        ],
    )


_prep1 = _make_prep(relu_sum=False, nx=1)
_prep2 = _make_prep(relu_sum=True, nx=NC)


def _gidx_body(j_ref, et_ref, g_ref):
    g_ref[...] = et_ref[...] * N2 + j_ref[...]


_gidx_call = pl.pallas_call(
    _gidx_body,
    out_shape=jax.ShapeDtypeStruct((E // 128, 128), jnp.int32),
)


def _fin_body(p_ref, out_ref):
    out_ref[...] = p_ref[0] + p_ref[1]


_fin_call = pl.pallas_call(
    _fin_body,
    grid=(NBLK,),
    in_specs=[pl.BlockSpec((NC, BN, D), lambda n: (0, n, 0))],
    out_specs=pl.BlockSpec((BN, D), lambda n: (n, 0)),
    out_shape=jax.ShapeDtypeStruct((N2, D), jnp.float32),
)


_sc_mesh = plsc.VectorSubcoreMesh(core_axis_name="c", subcore_axis_name="s")


@functools.partial(
    pl.kernel,
    out_type=jax.ShapeDtypeStruct((NC, N2, D), jnp.float32),
    mesh=_sc_mesh,
    scratch_types=[
        pltpu.VMEM_SHARED((N2, D), jnp.float32),
        pltpu.VMEM((3, CHUNK, D), jnp.float32),
        pltpu.VMEM((3, 1, CHUNK), jnp.int32),
        pltpu.VMEM((6, 1, CHUNK), jnp.int32),
        [pltpu.SemaphoreType.DMA] * 3,
        [pltpu.SemaphoreType.DMA] * 3,
        [pltpu.SemaphoreType.DMA] * 3,
        [pltpu.SemaphoreType.DMA] * 6,
    ],
)
def _sc_edges(y_hbm, gidx_hbm, didx_hbm, init_hbm, out_hbm,
              acc, rows, gbuf, dbuf, semG, semS, semIg, semId):
    cid = lax.axis_index("c")
    sid = lax.axis_index("s")
    base = (cid * NS + sid) * NCHUNK

    def g_at(j):
        return gbuf.at[j, 0]

    def d_at(j):
        return dbuf.at[j, 0]

    row0 = sid * ROWS_PT
    pltpu.sync_copy(init_hbm.at[cid, pl.ds(row0, ROWS_PT)],
                    acc.at[pl.ds(row0, ROWS_PT)])
    plsc.subcore_barrier()

    pltpu.sync_copy(gidx_hbm.at[base + 0], gbuf.at[0])
    pltpu.sync_copy(didx_hbm.at[base + 0], dbuf.at[0])
    pltpu.sync_copy(gidx_hbm.at[base + 1], gbuf.at[1])
    pltpu.sync_copy(didx_hbm.at[base + 1], dbuf.at[1])
    pltpu.async_copy(gidx_hbm.at[base + 2], gbuf.at[2], semIg[2])
    pltpu.async_copy(didx_hbm.at[base + 2], dbuf.at[2], semId[2])
    pltpu.async_copy(y_hbm.at[g_at(0)], rows.at[0], semG[0])
    pltpu.async_copy(y_hbm.at[g_at(1)], rows.at[1], semG[1])

    def do_slot(r6, i):
        c = 6 * r6 + i
        i3, i6 = i % 3, i % 6
        j3, j6 = (i + 2) % 3, (i + 3) % 6
        pltpu.make_async_copy(y_hbm.at[g_at(i3)], rows.at[i3],
                              semG[i3]).wait()
        pltpu.async_copy(rows.at[i3], acc.at[d_at(i6)], semS[i3], add=True)

        @pl.when(jnp.logical_and(c >= 1, c + 2 <= NCHUNK - 1))
        def _():
            pltpu.make_async_copy(rows.at[j3], acc.at[d_at((i + 5) % 6)],
                                  semS[j3]).wait()

        @pl.when(c + 2 <= NCHUNK - 1)
        def _():
            pltpu.make_async_copy(gidx_hbm.at[base], gbuf.at[j3],
                                  semIg[j3]).wait()
            pltpu.make_async_copy(didx_hbm.at[base], dbuf.at[(i + 2) % 6],
                                  semId[(i + 2) % 6]).wait()
            pltpu.async_copy(y_hbm.at[g_at(j3)], rows.at[j3], semG[j3])

        @pl.when(c + 3 <= NCHUNK - 1)
        def _():
            pltpu.async_copy(gidx_hbm.at[base + c + 3], gbuf.at[i3],
                             semIg[i3])
            pltpu.async_copy(didx_hbm.at[base + c + 3], dbuf.at[j6],
                             semId[j6])

    def round6(r6, carry):
        for i in range(6):
            do_slot(r6, i)
        return carry

    lax.fori_loop(0, NCHUNK // 6, round6, 0, unroll=False)
    for i in range(3):
        pltpu.make_async_copy(rows.at[i], acc.at[d_at(i)], semS[i]).wait()
    plsc.subcore_barrier()

    row0 = sid * ROWS_PT
    pltpu.sync_copy(acc.at[pl.ds(row0, ROWS_PT)],
                    out_hbm.at[cid, pl.ds(row0, ROWS_PT)])


def _layer(xin, gidx3, didx3, w, gwt, root, bias, relu_sum):
    prep = _prep2 if relu_sum else _prep1
    y, init = prep(xin, w, gwt, root, bias)
    return _sc_edges(y.reshape(R * N2, D), gidx3, didx3, init)


def kernel(meeting_utterance_enc_hidden_states, adj_coos, edge_types,
           basis1, att1, gate1, root1, bias1,
           basis2, att2, gate2, root2, bias2):
    x = jnp.pad(meeting_utterance_enc_hidden_states,
                ((0, 0), (0, N2 - N), (0, 0)))
    i_idx = adj_coos[0, 0]
    j_idx = adj_coos[0, 1]
    et = edge_types[0]

    gidx = _gidx_call(j_idx.reshape(E // 128, 128), et.reshape(E // 128, 128))
    pad_dst = N + jnp.arange(E2 - E, dtype=jnp.int32) % (N2 - N)
    pad_src = jnp.arange(E2 - E, dtype=jnp.int32) % N2
    gidx3 = jnp.concatenate([gidx.reshape(E),
                             pad_src]).reshape(NW * NCHUNK, 1, CHUNK)
    didx3 = jnp.concatenate([i_idx, pad_dst]).reshape(NW * NCHUNK, 1, CHUNK)

    w1 = _w_call(att1, basis1.reshape(NB, D * D)).reshape(R, D, D)
    w2 = _w_call(att2, basis2.reshape(NB, D * D)).reshape(R, D, D)
    gwt1 = gate1[:, :, 0].T
    gwt2 = gate2[:, :, 0].T
    bias1_2d = bias1.reshape(1, D)
    bias2_2d = bias2.reshape(1, D)

    p1 = _layer(x, gidx3, didx3, w1, gwt1, root1, bias1_2d, relu_sum=False)
    p2 = _layer(p1, gidx3, didx3, w2, gwt2, root2, bias2_2d, relu_sum=True)
    return _fin_call(p2)[:N]


# speedup vs baseline: 1.1226x; 1.0211x over previous
import functools

import jax
import jax.numpy as jnp
from jax import lax
from jax.experimental import pallas as pl
from jax.experimental.pallas import tpu as pltpu
from jax.experimental.pallas import tpu_sc as plsc

N = 10000
N2 = 10240
D = 128
E = 320000
R = 2
NB = 30

NC = 2
NS = 16
NW = NC * NS
CHUNK = 120
NCHUNK = 84
E2 = NW * NCHUNK * CHUNK
ROWS_PT = N2 // NS
STAGES = (120, 120, 120, 120, 120, 40)

BN = 1024
NBLK = N2 // BN


def _w_body(att_ref, basis_ref, w_ref):
    w_ref[...] = jax.lax.dot_general(
        att_ref[...], basis_ref[...], (((1,), (0,)), ((), ())),
        preferred_element_type=jnp.float32, precision=lax.Precision.HIGHEST)


_w_call = pl.pallas_call(
    _w_body,
    out_shape=jax.ShapeDtypeStruct((R, D * D), jnp.float32),
)


def _prep_body(xin_ref, w_ref, gwt_ref, root_ref, bias_ref, y_ref, init_ref,
               *, relu_sum):
    if relu_sum:
        x = jax.nn.relu(xin_ref[0] + xin_ref[1])
    else:
        x = xin_ref[0]
    dot = functools.partial(
        jax.lax.dot_general, dimension_numbers=(((1,), (0,)), ((), ())),
        preferred_element_type=jnp.float32, precision=lax.Precision.HIGHEST)
    s = jax.nn.sigmoid(dot(x, gwt_ref[...]))
    for r in range(R):
        y_ref[r] = s[:, r:r + 1] * dot(x, w_ref[r])
    init_ref[0] = dot(x, root_ref[...]) + bias_ref[...]
    init_ref[1] = jnp.zeros((BN, D), jnp.float32)


def _make_prep(relu_sum, nx):
    return pl.pallas_call(
        functools.partial(_prep_body, relu_sum=relu_sum),
        grid=(NBLK,),
        in_specs=[
            pl.BlockSpec((nx, BN, D), lambda n: (0, n, 0)),
            pl.BlockSpec((R, D, D), lambda n: (0, 0, 0)),
            pl.BlockSpec((D, R), lambda n: (0, 0)),
            pl.BlockSpec((D, D), lambda n: (0, 0)),
            pl.BlockSpec((1, D), lambda n: (0, 0)),
        ],
        out_specs=[
            pl.BlockSpec((R, BN, D), lambda n: (0, n, 0)),
            pl.BlockSpec((NC, BN, D), lambda n: (0, n, 0)),
        ],
        out_shape=[
            jax.ShapeDtypeStruct((R, N2, D), jnp.float32),
            jax.ShapeDtypeStruct((NC, N2, D), jnp.float32),
        ],
    )


_prep1 = _make_prep(relu_sum=False, nx=1)
_prep2 = _make_prep(relu_sum=True, nx=NC)


def _gidx_body(j_ref, et_ref, g_ref):
    g_ref[...] = et_ref[...] * N2 + j_ref[...]


_gidx_call = pl.pallas_call(
    _gidx_body,
    out_shape=jax.ShapeDtypeStruct((E // 128, 128), jnp.int32),
)


def _fin_body(p_ref, out_ref):
    out_ref[...] = p_ref[0] + p_ref[1]


_FB = 2000
_fin_call = pl.pallas_call(
    _fin_body,
    grid=(N // _FB,),
    in_specs=[pl.BlockSpec((NC, _FB, D), lambda n: (0, n, 0))],
    out_specs=pl.BlockSpec((_FB, D), lambda n: (n, 0)),
    out_shape=jax.ShapeDtypeStruct((N, D), jnp.float32),
)


_sc_mesh = plsc.VectorSubcoreMesh(core_axis_name="c", subcore_axis_name="s")


@functools.partial(
    pl.kernel,
    out_type=jax.ShapeDtypeStruct((NC, N2, D), jnp.float32),
    mesh=_sc_mesh,
    scratch_types=[
        pltpu.VMEM_SHARED((N2, D), jnp.float32),
        pltpu.VMEM((3, CHUNK, D), jnp.float32),
        pltpu.VMEM((3, 1, CHUNK), jnp.int32),
        pltpu.VMEM((6, 1, CHUNK), jnp.int32),
        [pltpu.SemaphoreType.DMA] * 3,
        [pltpu.SemaphoreType.DMA] * 3,
        [pltpu.SemaphoreType.DMA] * 3,
        [pltpu.SemaphoreType.DMA] * 6,
    ],
)
def _sc_edges(y_hbm, gidx_hbm, didx_hbm, init_hbm, out_hbm,
              acc, rows, gbuf, dbuf, semG, semS, semIg, semId):
    cid = lax.axis_index("c")
    sid = lax.axis_index("s")
    base = (cid * NS + sid) * NCHUNK

    def g_at(j):
        return gbuf.at[j, 0]

    def d_at(j):
        return dbuf.at[j, 0]

    row0 = sid * ROWS_PT
    pltpu.sync_copy(init_hbm.at[cid, pl.ds(row0, ROWS_PT)],
                    acc.at[pl.ds(row0, ROWS_PT)])
    plsc.subcore_barrier()

    pltpu.sync_copy(gidx_hbm.at[base + 0], gbuf.at[0])
    pltpu.sync_copy(didx_hbm.at[base + 0], dbuf.at[0])
    pltpu.sync_copy(gidx_hbm.at[base + 1], gbuf.at[1])
    pltpu.sync_copy(didx_hbm.at[base + 1], dbuf.at[1])
    pltpu.async_copy(gidx_hbm.at[base + 2], gbuf.at[2], semIg[2])
    pltpu.async_copy(didx_hbm.at[base + 2], dbuf.at[2], semId[2])
    pltpu.async_copy(y_hbm.at[g_at(0)], rows.at[0], semG[0])
    pltpu.async_copy(y_hbm.at[g_at(1)], rows.at[1], semG[1])

    def do_slot(r6, i):
        c = 6 * r6 + i
        i3, i6 = i % 3, i % 6
        j3, j6 = (i + 2) % 3, (i + 3) % 6
        pltpu.make_async_copy(y_hbm.at[g_at(i3)], rows.at[i3],
                              semG[i3]).wait()
        pltpu.async_copy(rows.at[i3], acc.at[d_at(i6)], semS[i3], add=True)

        @pl.when(jnp.logical_and(c >= 1, c + 2 <= NCHUNK - 1))
        def _():
            pltpu.make_async_copy(rows.at[j3], acc.at[d_at((i + 5) % 6)],
                                  semS[j3]).wait()

        @pl.when(c + 2 <= NCHUNK - 1)
        def _():
            pltpu.make_async_copy(gidx_hbm.at[base], gbuf.at[j3],
                                  semIg[j3]).wait()
            pltpu.make_async_copy(didx_hbm.at[base], dbuf.at[(i + 2) % 6],
                                  semId[(i + 2) % 6]).wait()
            pltpu.async_copy(y_hbm.at[g_at(j3)], rows.at[j3], semG[j3])

        @pl.when(c + 3 <= NCHUNK - 1)
        def _():
            pltpu.async_copy(gidx_hbm.at[base + c + 3], gbuf.at[i3],
                             semIg[i3])
            pltpu.async_copy(didx_hbm.at[base + c + 3], dbuf.at[j6],
                             semId[j6])

    def round6(r6, carry):
        for i in range(6):
            do_slot(r6, i)
        return carry

    lax.fori_loop(0, NCHUNK // 6, round6, 0, unroll=False)
    for i in range(3):
        pltpu.make_async_copy(rows.at[i], acc.at[d_at(i)], semS[i]).wait()
    plsc.subcore_barrier()

    row0 = sid * ROWS_PT
    pltpu.sync_copy(acc.at[pl.ds(row0, ROWS_PT)],
                    out_hbm.at[cid, pl.ds(row0, ROWS_PT)])


def _layer(xin, gidx3, didx3, w, gwt, root, bias, relu_sum):
    prep = _prep2 if relu_sum else _prep1
    y, init = prep(xin, w, gwt, root, bias)
    return _sc_edges(y.reshape(R * N2, D), gidx3, didx3, init)


def kernel(meeting_utterance_enc_hidden_states, adj_coos, edge_types,
           basis1, att1, gate1, root1, bias1,
           basis2, att2, gate2, root2, bias2):
    x = jnp.pad(meeting_utterance_enc_hidden_states,
                ((0, 0), (0, N2 - N), (0, 0)))
    i_idx = adj_coos[0, 0]
    j_idx = adj_coos[0, 1]
    et = edge_types[0]

    gidx = _gidx_call(j_idx.reshape(E // 128, 128), et.reshape(E // 128, 128))
    pad_dst = N + jnp.arange(E2 - E, dtype=jnp.int32) % (N2 - N)
    pad_src = jnp.arange(E2 - E, dtype=jnp.int32) % N2
    gidx3 = jnp.concatenate([gidx.reshape(E),
                             pad_src]).reshape(NW * NCHUNK, 1, CHUNK)
    didx3 = jnp.concatenate([i_idx, pad_dst]).reshape(NW * NCHUNK, 1, CHUNK)

    w1 = _w_call(att1, basis1.reshape(NB, D * D)).reshape(R, D, D)
    w2 = _w_call(att2, basis2.reshape(NB, D * D)).reshape(R, D, D)
    gwt1 = gate1[:, :, 0].T
    gwt2 = gate2[:, :, 0].T
    bias1_2d = bias1.reshape(1, D)
    bias2_2d = bias2.reshape(1, D)

    p1 = _layer(x, gidx3, didx3, w1, gwt1, root1, bias1_2d, relu_sum=False)
    p2 = _layer(p1, gidx3, didx3, w2, gwt2, root2, bias2_2d, relu_sum=True)
    return _fin_call(p2)
